# baseline (speedup 1.0000x reference)
import jax
import jax.numpy as jnp
from jax.experimental import pallas as pl
from jax.experimental.pallas import tpu as pltpu


def _s1_kernel(x_ref, w1_ref, o_ref):
    o_ref[...] = jnp.dot(x_ref[...], w1_ref[...],
                         preferred_element_type=jnp.float32)


def _fused_kernel(G, B, adj_ref, s1_ref, b1_ref, w2_ref, b2_ref, o_ref,
                  s2_ref, z_ref):
    t = pl.program_id(0)

    @pl.when(t < G)
    def _phase0():
        h = jnp.dot(adj_ref[...], s1_ref[...],
                    preferred_element_type=jnp.float32) + b1_ref[...]
        h = jnp.maximum(h, 0.0)
        s2_ref[pl.ds(t * B, B), :] = jnp.dot(
            h, w2_ref[...], preferred_element_type=jnp.float32)

    @pl.when((t >= G) & (t < 2 * G))
    def _phase1():
        i = t - G
        z_ref[pl.ds(i * B, B), :] = jnp.dot(
            adj_ref[...], s2_ref[...],
            preferred_element_type=jnp.float32) + b2_ref[...]

    @pl.when(t >= 2 * G)
    def _phase2():
        i = t - 2 * G
        zi = z_ref[pl.ds(i * B, B), :]
        zz = jax.lax.dot_general(
            zi, z_ref[...], (((1,), (1,)), ((), ())),
            preferred_element_type=jnp.float32)
        o_ref[...] = jax.nn.sigmoid(zz)


def kernel(x, adj, W1, b1, W2, b2):
    N, F = x.shape
    H = W1.shape[1]
    C = W2.shape[1]
    b1r = b1.reshape(1, H)
    b2r = b2.reshape(1, C)

    s1 = pl.pallas_call(
        _s1_kernel,
        out_shape=jax.ShapeDtypeStruct((N, H), jnp.float32),
    )(x, W1)

    B = 200
    G = N // B

    def adj_map(t):
        return (jnp.where(t < 2 * G, t % G, G - 1), 0)

    def out_map(t):
        return (jnp.where(t < 2 * G, 0, t - 2 * G), 0)

    fused = lambda *refs: _fused_kernel(G, B, *refs)
    a_pred = pl.pallas_call(
        fused,
        grid=(3 * G,),
        in_specs=[
            pl.BlockSpec((B, N), adj_map),
            pl.BlockSpec((N, H), lambda t: (0, 0)),
            pl.BlockSpec((1, H), lambda t: (0, 0)),
            pl.BlockSpec((H, C), lambda t: (0, 0)),
            pl.BlockSpec((1, C), lambda t: (0, 0)),
        ],
        out_specs=pl.BlockSpec((B, N), out_map),
        out_shape=jax.ShapeDtypeStruct((N, N), jnp.float32),
        scratch_shapes=[
            pltpu.VMEM((N, C), jnp.float32),
            pltpu.VMEM((N, C), jnp.float32),
        ],
    )(adj, s1, b1r, W2, b2r)
    return a_pred


# speedup vs baseline: 1.0211x; 1.0211x over previous
import jax
import jax.numpy as jnp
from jax.experimental import pallas as pl
from jax.experimental.pallas import tpu as pltpu


def _s1_kernel(x_ref, w1_ref, o_ref):
    o_ref[...] = jnp.dot(x_ref[...], w1_ref[...],
                         preferred_element_type=jnp.float32)


def _fused_kernel(GA, BA, GD, BD, C, adj_ref, s1_ref, b1_ref, w2_ref,
                  b2_ref, o_ref, sz_ref):
    t = pl.program_id(0)

    @pl.when(t < GA)
    def _phase0():
        h = jnp.dot(adj_ref[...], s1_ref[...],
                    preferred_element_type=jnp.float32) + b1_ref[...]
        h = jnp.maximum(h, 0.0)
        sz_ref[pl.ds(t * BA, BA), :C] = jnp.dot(
            h, w2_ref[...], preferred_element_type=jnp.float32)

    @pl.when((t >= GA) & (t < 2 * GA))
    def _phase1():
        i = t - GA
        sz_ref[pl.ds(i * BA, BA), C:] = jnp.dot(
            adj_ref[...], sz_ref[:, :C],
            preferred_element_type=jnp.float32) + b2_ref[...]

    @pl.when(t >= 2 * GA)
    def _phase2():
        i = t - 2 * GA
        zi = sz_ref[pl.ds(i * BD, BD), C:]
        zz = jax.lax.dot_general(
            zi, sz_ref[:, C:], (((1,), (1,)), ((), ())),
            preferred_element_type=jnp.float32)
        o_ref[...] = jax.nn.sigmoid(zz)


def kernel(x, adj, W1, b1, W2, b2):
    N, F = x.shape
    H = W1.shape[1]
    C = W2.shape[1]
    b1r = b1.reshape(1, H)
    b2r = b2.reshape(1, C)

    s1 = pl.pallas_call(
        _s1_kernel,
        out_shape=jax.ShapeDtypeStruct((N, H), jnp.float32),
    )(x, W1)

    BA = 400
    GA = N // BA
    BD = 200
    GD = N // BD
    T = 2 * GA + GD

    def adj_map(t):
        return (jnp.where(t < 2 * GA, t % GA, GA - 1), 0)

    def out_map(t):
        return (jnp.where(t < 2 * GA, 0, t - 2 * GA), 0)

    fused = lambda *refs: _fused_kernel(GA, BA, GD, BD, C, *refs)
    a_pred = pl.pallas_call(
        fused,
        grid=(T,),
        in_specs=[
            pl.BlockSpec((BA, N), adj_map),
            pl.BlockSpec((N, H), lambda t: (0, 0)),
            pl.BlockSpec((1, H), lambda t: (0, 0)),
            pl.BlockSpec((H, C), lambda t: (0, 0)),
            pl.BlockSpec((1, C), lambda t: (0, 0)),
        ],
        out_specs=pl.BlockSpec((BD, N), out_map),
        out_shape=jax.ShapeDtypeStruct((N, N), jnp.float32),
        scratch_shapes=[
            pltpu.VMEM((N, 2 * C), jnp.float32),
        ],
    )(adj, s1, b1r, W2, b2r)
    return a_pred
